# baseline (speedup 1.0000x reference)
import functools

import numpy as np

import jax
import jax.numpy as jnp
from jax import lax
from jax.experimental import pallas as pl
from jax.experimental.layout import Format, Layout
from jax.experimental.pallas import tpu as pltpu
from jax.experimental.pallas import tpu_sc as plsc

_ROWS = 4096
_COLS = 200
_D = 21
_N = _ROWS * _COLS
_NC, _NS, _L = 2, 16, 16
_NW = _NC * _NS
_IB = _ROWS // _NW
_JT = _COLS // 8
_NV = _IB // _L

_HALF_OF = {2: 22, 11: 22, 3: 23, 13: 23, 7: 24, 9: 24}


def _body(seq_hbm, out_hbm, seq_v, q_v, ssem):
    wid = lax.axis_index("s") * _NC + lax.axis_index("c")
    i0 = wid * _IB
    pltpu.sync_copy(seq_hbm.at[pl.ds(i0, _IB)], seq_v)

    iota16 = lax.iota(jnp.int32, _L)

    def chunk_body(jt, qb):
        @pl.when(jt >= 2)
        def _():
            for k in range(_D):
                pltpu.make_async_copy(
                    q_v.at[qb, k], out_hbm.at[k, jt - 2, wid], ssem,
                ).wait()

        @plsc.parallel_loop(0, 8, unroll=2)
        def j_step(jj):
            j = jt * 8 + jj
            jvec = jnp.broadcast_to(j, (_L,))
            vals = [
                plsc.load_gather(seq_v, [iota16 + r * _L, jvec])
                for r in range(_NV)
            ]
            for k in range(_D):
                h = _HALF_OF.get(k)
                for r in range(_NV):
                    hit = vals[r] == (k + 1)
                    if h is None:
                        ov = jnp.where(hit, 1.0, 0.0).astype(jnp.float32)
                    else:
                        ov = jnp.where(
                            hit, 1.0,
                            jnp.where(vals[r] == h, 0.5, 0.0)
                        ).astype(jnp.float32)
                    q_v[qb, k, jj, pl.ds(r * _L, _L)] = ov

        for k in range(_D):
            pltpu.make_async_copy(
                q_v.at[qb, k], out_hbm.at[k, jt, wid], ssem,
            ).start()

    def pair_step(p, carry):
        chunk_body(2 * p, 0)
        chunk_body(2 * p + 1, 1)
        return carry

    lax.fori_loop(0, _JT // 2, pair_step, 0)
    chunk_body(_JT - 1, 0)

    for jt in (_JT - 2, _JT - 1):
        for k in range(_D):
            pltpu.make_async_copy(
                q_v.at[jt % 2, k], out_hbm.at[k, jt, wid], ssem,
            ).wait()


def _run(sequence):
    seq = sequence.astype(jnp.int32)
    lookup = functools.partial(
        pl.kernel,
        out_type=jax.ShapeDtypeStruct((_D, _JT, _NW, 8, 128), jnp.float32),
        mesh=plsc.VectorSubcoreMesh(core_axis_name="c", subcore_axis_name="s"),
        compiler_params=pltpu.CompilerParams(needs_layout_passes=False),
        scratch_types=[
            pltpu.VMEM((_IB, _COLS), jnp.int32),
            pltpu.VMEM((2, _D, 8, 128), jnp.float32),
            pltpu.SemaphoreType.DMA,
        ],
    )(_body)
    out_phys = lookup(seq)
    out = jnp.transpose(out_phys, (2, 4, 1, 3, 0))
    return out.reshape(_ROWS, _COLS, _D)


_jitted = None


def kernel(sequence):
    global _jitted
    if isinstance(sequence, jax.core.Tracer):
        return _run(sequence)
    if _jitted is None:
        try:
            sharding = sequence.sharding
        except AttributeError:
            sharding = jax.sharding.SingleDeviceSharding(jax.devices()[0])
        fmt = Format(Layout(major_to_minor=(2, 1, 0)), sharding)
        _jitted = jax.jit(_run, out_shardings=fmt)
    return _jitted(sequence)


# speedup vs baseline: 1.1307x; 1.1307x over previous
import functools

import numpy as np

import jax
import jax.numpy as jnp
from jax import lax
from jax.experimental import pallas as pl
from jax.experimental.layout import Format, Layout
from jax.experimental.pallas import tpu as pltpu
from jax.experimental.pallas import tpu_sc as plsc

_ROWS = 4096
_COLS = 200
_D = 21
_N = _ROWS * _COLS
_NC, _NS, _L = 2, 16, 16
_NW = _NC * _NS
_IB = _ROWS // _NW
_JT = _COLS // 8
_NV = _IB // _L

_HALF_OF = {2: 22, 11: 22, 3: 23, 13: 23, 7: 24, 9: 24}


def _body(seq_hbm, out_hbm, seq_v, q_v, ssem):
    wid = lax.axis_index("s") * _NC + lax.axis_index("c")
    i0 = wid * _IB
    pltpu.sync_copy(seq_hbm.at[pl.ds(i0 * _COLS, _IB * _COLS)], seq_v)

    stride_i = lax.iota(jnp.int32, _L) * _COLS

    def chunk_step(jt, carry):
        qb = lax.rem(jt, 2)

        @pl.when(jt >= 2)
        def _():
            for k in range(_D):
                pltpu.make_async_copy(
                    q_v.at[qb, k], out_hbm.at[k, jt - 2, wid], ssem,
                ).wait()

        def j_step(jj, carry2):
            j = jt * 8 + jj
            vals = [
                plsc.load_gather(seq_v, [stride_i + (r * _L * _COLS + j)])
                for r in range(_NV)
            ]
            for k in range(_D):
                h = _HALF_OF.get(k)
                for r in range(_NV):
                    hit = vals[r] == (k + 1)
                    if h is None:
                        ov = jnp.where(hit, 1.0, 0.0).astype(jnp.float32)
                    else:
                        ov = jnp.where(
                            hit, 1.0,
                            jnp.where(vals[r] == h, 0.5, 0.0)
                        ).astype(jnp.float32)
                    q_v[qb, k, jj, pl.ds(r * _L, _L)] = ov
            return carry2

        lax.fori_loop(0, 8, j_step, 0)

        for k in range(_D):
            pltpu.make_async_copy(
                q_v.at[qb, k], out_hbm.at[k, jt, wid], ssem,
            ).start()
        return carry

    lax.fori_loop(0, _JT, chunk_step, 0)

    for jt in (_JT - 2, _JT - 1):
        for k in range(_D):
            pltpu.make_async_copy(
                q_v.at[jt % 2, k], out_hbm.at[k, jt, wid], ssem,
            ).wait()


def _run(sequence):
    seq = sequence.astype(jnp.int32).reshape(_N)
    lookup = functools.partial(
        pl.kernel,
        out_type=jax.ShapeDtypeStruct((_D, _JT, _NW, 8, 128), jnp.float32),
        mesh=plsc.VectorSubcoreMesh(core_axis_name="c", subcore_axis_name="s"),
        compiler_params=pltpu.CompilerParams(
            use_tc_tiling_on_sc=False, needs_layout_passes=False),
        scratch_types=[
            pltpu.VMEM((_IB * _COLS,), jnp.int32),
            pltpu.VMEM((2, _D, 8, 128), jnp.float32),
            pltpu.SemaphoreType.DMA,
        ],
    )(_body)
    out_phys = lookup(seq)
    out = jnp.transpose(out_phys, (2, 4, 1, 3, 0))
    return out.reshape(_ROWS, _COLS, _D)


_jitted = None


def kernel(sequence):
    global _jitted
    if isinstance(sequence, jax.core.Tracer):
        return _run(sequence)
    if _jitted is None:
        try:
            sharding = sequence.sharding
        except AttributeError:
            sharding = jax.sharding.SingleDeviceSharding(jax.devices()[0])
        fmt = Format(Layout(major_to_minor=(2, 1, 0)), sharding)
        _jitted = jax.jit(_run, out_shardings=fmt)
    return _jitted(sequence)
